# baseline (speedup 1.0000x reference)
import functools

import jax
import jax.numpy as jnp
from jax import lax
from jax.experimental import pallas as pl
from jax.experimental.pallas import tpu as pltpu
from jax.experimental.pallas import tpu_sc as plsc

_N = 4096
_D = 128
_HW = 144
_SINK = 8

_CH = 128
_NCH = _N * 16 // (16 * _CH)
_RPS = _N // 16


def _enc_body(x_ref, w_ref, b_ref, o_ref):
    h = jnp.dot(x_ref[...], w_ref[...], preferred_element_type=jnp.float32)
    h = jnp.maximum(h + b_ref[...], 0.0)
    o_ref[:, :_D] = h
    lane = jax.lax.broadcasted_iota(jnp.int32, (h.shape[0], _HW - _D), 1)
    o_ref[:, _D:] = jnp.where(lane == 0, 1.0, 0.0)


def _encode(x_all, W_enc, b_enc):
    nb = 8
    rb = (2 * _N) // nb
    return pl.pallas_call(
        _enc_body,
        grid=(nb,),
        in_specs=[
            pl.BlockSpec((rb, _D), lambda i: (i, 0)),
            pl.BlockSpec((_D, _D), lambda i: (0, 0)),
            pl.BlockSpec((1, _D), lambda i: (0, 0)),
        ],
        out_specs=pl.BlockSpec((rb, _HW), lambda i: (i, 0)),
        out_shape=jax.ShapeDtypeStruct((2 * _N, _HW), jnp.float32),
    )(x_all, W_enc, b_enc.reshape(1, _D))


def _seg_body(hext, src, dst, zrow, out, src_v, dst_v, rows_v, acc, sem):
    c = lax.axis_index("c")
    s = lax.axis_index("s")
    pltpu.sync_copy(zrow, acc.at[pl.ds(s * _RPS, _RPS)])
    pltpu.sync_copy(src.at[c, s], src_v)
    pltpu.sync_copy(dst.at[c, s], dst_v)
    plsc.subcore_barrier()

    def chunk(j, carry):
        pltpu.async_copy(hext.at[src_v.at[j]], rows_v, sem).wait()
        pltpu.sync_copy(rows_v, acc.at[dst_v.at[j]], add=True)
        return carry

    lax.fori_loop(0, _NCH, chunk, 0)
    plsc.subcore_barrier()
    pltpu.sync_copy(acc.at[pl.ds(s * _RPS, _RPS)],
                    out.at[c, pl.ds(s * _RPS, _RPS)])


def _segment_sum_sc(h_ext, src_r, dst_r, zrow):
    return pl.kernel(
        _seg_body,
        out_type=jax.ShapeDtypeStruct((2, _N, _HW), jnp.float32),
        mesh=plsc.VectorSubcoreMesh(core_axis_name="c", subcore_axis_name="s"),
        scratch_types=[
            pltpu.VMEM((_NCH, _CH), jnp.int32),
            pltpu.VMEM((_NCH, _CH), jnp.int32),
            pltpu.VMEM((_CH, _HW), jnp.float32),
            pltpu.VMEM_SHARED((_N, _HW), jnp.float32),
            pltpu.SemaphoreType.DMA,
        ],
        compiler_params=pltpu.CompilerParams(use_tc_tiling_on_sc=False),
    )(h_ext, src_r, dst_r, zrow)


def _mix_body(hext_ref, p0_ref, wa_ref, wb_ref, b_ref, s_ref, o_ref):
    acc = p0_ref[...]
    deg = jnp.clip(acc[:, _D:_D + 1], 1.0, None)
    msg = acc[:, :_D] / deg
    h = hext_ref[:, :_D]
    o = jnp.dot(h, wa_ref[...], preferred_element_type=jnp.float32)
    o = o + jnp.dot(msg, wb_ref[...], preferred_element_type=jnp.float32)
    o = jnp.maximum(o + b_ref[...], 0.0)
    nrm = jnp.sqrt(jnp.sum(o * o, axis=1, keepdims=True)) + 1e-8
    o_ref[...] = (o * (s_ref[0, 0] / nrm)).astype(jnp.bfloat16)


def _mix(h_ext, part0, W_msg, b_msg, inv_sqrt_lambda):
    nb = 8
    rb = (2 * _N) // nb
    return pl.pallas_call(
        _mix_body,
        grid=(nb,),
        in_specs=[
            pl.BlockSpec((rb, _HW), lambda i: (i, 0)),
            pl.BlockSpec((rb, _HW), lambda i: (i, 0)),
            pl.BlockSpec((_D, _D), lambda i: (0, 0)),
            pl.BlockSpec((_D, _D), lambda i: (0, 0)),
            pl.BlockSpec((1, _D), lambda i: (0, 0)),
            pl.BlockSpec(memory_space=pltpu.SMEM),
        ],
        out_specs=pl.BlockSpec((rb, _D), lambda i: (i, 0)),
        out_shape=jax.ShapeDtypeStruct((2 * _N, _D), jnp.bfloat16),
    )(h_ext, part0, W_msg[:_D], W_msg[_D:], b_msg.reshape(1, _D),
      inv_sqrt_lambda)


_RB = 256
_NBLK = _N // _RB
_G = 16
_Q = 1024


def _sink_body(feats_ref, out_ref, K_ref, u_ref, v_ref):
    i = pl.program_id(0)

    @pl.when(i == 0)
    def _build_and_iterate():
        dT = feats_ref[pl.ds(_N, _N), :].T

        def build(ib, _):
            t = feats_ref[pl.ds(ib * _RB, _RB), :]
            corr = jnp.dot(t, dT, preferred_element_type=jnp.float32)
            Kb = jnp.exp(corr)
            K_ref[pl.ds(ib * _RB, _RB), :] = Kb.astype(jnp.bfloat16)
            rs = jnp.sum(Kb, axis=1, keepdims=True)
            u_ref[pl.ds(ib * _RB, _RB), :] = 0.9 / (rs + 1e-8)
            return 0

        jax.lax.fori_loop(0, _NBLK, build, 0)

        def update_v(_):
            for q in range(_N // _Q):
                def gbody(g, acc):
                    Kb = K_ref[pl.ds(g * _G, _G),
                               q * _Q:(q + 1) * _Q].astype(jnp.float32)
                    ub = u_ref[pl.ds(g * _G, _G), :]
                    return acc + Kb * ub

                acc = jax.lax.fori_loop(0, _N // _G, gbody,
                                        jnp.zeros((_G, _Q), jnp.float32))
                z = jnp.sum(acc, axis=0, keepdims=True)
                v_ref[:, q * _Q:(q + 1) * _Q] = 0.9 / (z + 1e-8)

        def update_u(_):
            def gbody(g, _c):
                tot = jnp.zeros((_G, 1), jnp.float32)
                for q in range(_N // _Q):
                    Kb = K_ref[pl.ds(g * _G, _G),
                               q * _Q:(q + 1) * _Q].astype(jnp.float32)
                    vS = v_ref[:, q * _Q:(q + 1) * _Q]
                    tot = tot + jnp.sum(Kb * vS, axis=1, keepdims=True)
                u_ref[pl.ds(g * _G, _G), :] = 0.9 / (tot + 1e-8)
                return 0

            jax.lax.fori_loop(0, _N // _G, gbody, 0)

        def sink_iter(t, _):
            update_v(None)

            @pl.when(t < _SINK - 1)
            def _():
                update_u(None)

            return 0

        jax.lax.fori_loop(0, _SINK, sink_iter, 0)

    Kb = K_ref[pl.ds(i * _RB, _RB), :].astype(jnp.float32)
    out_ref[...] = u_ref[pl.ds(i * _RB, _RB), :] * Kb * v_ref[...]


def _sinkhorn(feats):
    return pl.pallas_call(
        _sink_body,
        grid=(_NBLK,),
        in_specs=[pl.BlockSpec((2 * _N, _D), lambda i: (0, 0))],
        out_specs=pl.BlockSpec((_RB, _N), lambda i: (i, 0)),
        out_shape=jax.ShapeDtypeStruct((_N, _N), jnp.float32),
        scratch_shapes=[
            pltpu.VMEM((_N, _N), jnp.bfloat16),
            pltpu.VMEM((_N, 1), jnp.float32),
            pltpu.VMEM((1, _N), jnp.float32),
        ],
        compiler_params=pltpu.CompilerParams(
            dimension_semantics=("arbitrary",),
            vmem_limit_bytes=64 * 1024 * 1024,
        ),
    )(feats)


def kernel(tra_x, tra_edge_index, tra_batch, det_x, det_edge_index,
           det_batch, W_enc, b_enc, W_msg, b_msg, eplison):
    x_all = jnp.concatenate([tra_x, det_x], axis=0)
    src_r = jnp.concatenate([tra_edge_index[0], det_edge_index[0] + _N]
                            ).reshape(2, 16, _NCH, _CH)
    dst_r = jnp.concatenate([tra_edge_index[1], det_edge_index[1]]
                            ).reshape(2, 16, _NCH, _CH)
    zrow = jnp.zeros((_RPS, _HW), jnp.float32)

    h_ext = _encode(x_all, W_enc, b_enc)

    acc = _segment_sum_sc(h_ext, src_r, dst_r, zrow).reshape(2 * _N, _HW)

    lambd = jnp.exp(eplison[0]) + 0.03
    inv_sqrt_lambda = (1.0 / jnp.sqrt(lambd)).reshape(1, 1)

    feats = _mix(h_ext, acc, W_msg, b_msg, inv_sqrt_lambda)
    return _sinkhorn(feats)


# speedup vs baseline: 3.1729x; 3.1729x over previous
import functools

import jax
import jax.numpy as jnp
from jax import lax
from jax.experimental import pallas as pl
from jax.experimental.pallas import tpu as pltpu
from jax.experimental.pallas import tpu_sc as plsc

_N = 4096
_D = 128
_HW = 144
_SINK = 8

_CH = 128
_NCH = _N * 16 // (16 * _CH)
_RPS = _N // 16


def _enc_body(x_ref, w_ref, b_ref, o_ref):
    h = jnp.dot(x_ref[...], w_ref[...], preferred_element_type=jnp.float32)
    h = jnp.maximum(h + b_ref[...], 0.0)
    o_ref[:, :_D] = h
    lane = jax.lax.broadcasted_iota(jnp.int32, (h.shape[0], _HW - _D), 1)
    o_ref[:, _D:] = jnp.where(lane == 0, 1.0, 0.0)


def _encode(x_all, W_enc, b_enc):
    nb = 8
    rb = (2 * _N) // nb
    return pl.pallas_call(
        _enc_body,
        grid=(nb,),
        in_specs=[
            pl.BlockSpec((rb, _D), lambda i: (i, 0)),
            pl.BlockSpec((_D, _D), lambda i: (0, 0)),
            pl.BlockSpec((1, _D), lambda i: (0, 0)),
        ],
        out_specs=pl.BlockSpec((rb, _HW), lambda i: (i, 0)),
        out_shape=jax.ShapeDtypeStruct((2 * _N, _HW), jnp.float32),
    )(x_all, W_enc, b_enc.reshape(1, _D))


def _seg_body(hext, src, dst, zrow, out, src_v, dst_v, rows_v, acc, sem):
    c = lax.axis_index("c")
    s = lax.axis_index("s")
    pltpu.sync_copy(zrow, acc.at[pl.ds(s * _RPS, _RPS)])
    pltpu.sync_copy(src.at[c, s], src_v)
    pltpu.sync_copy(dst.at[c, s], dst_v)
    plsc.subcore_barrier()

    def chunk(j, carry):
        pltpu.async_copy(hext.at[src_v.at[j]], rows_v, sem).wait()
        pltpu.sync_copy(rows_v, acc.at[dst_v.at[j]], add=True)
        return carry

    lax.fori_loop(0, _NCH, chunk, 0)
    plsc.subcore_barrier()
    pltpu.sync_copy(acc.at[pl.ds(s * _RPS, _RPS)],
                    out.at[c, pl.ds(s * _RPS, _RPS)])


def _segment_sum_sc(h_ext, src_r, dst_r, zrow):
    return pl.kernel(
        _seg_body,
        out_type=jax.ShapeDtypeStruct((2, _N, _HW), jnp.float32),
        mesh=plsc.VectorSubcoreMesh(core_axis_name="c", subcore_axis_name="s"),
        scratch_types=[
            pltpu.VMEM((_NCH, _CH), jnp.int32),
            pltpu.VMEM((_NCH, _CH), jnp.int32),
            pltpu.VMEM((_CH, _HW), jnp.float32),
            pltpu.VMEM_SHARED((_N, _HW), jnp.float32),
            pltpu.SemaphoreType.DMA,
        ],
        compiler_params=pltpu.CompilerParams(use_tc_tiling_on_sc=False),
    )(h_ext, src_r, dst_r, zrow)


def _mix_body(hext_ref, p0_ref, wa_ref, wb_ref, b_ref, s_ref, o_ref):
    acc = p0_ref[...]
    deg = jnp.clip(acc[:, _D:_D + 1], 1.0, None)
    msg = acc[:, :_D] / deg
    h = hext_ref[:, :_D]
    o = jnp.dot(h, wa_ref[...], preferred_element_type=jnp.float32)
    o = o + jnp.dot(msg, wb_ref[...], preferred_element_type=jnp.float32)
    o = jnp.maximum(o + b_ref[...], 0.0)
    nrm = jnp.sqrt(jnp.sum(o * o, axis=1, keepdims=True)) + 1e-8
    o_ref[...] = (o * (s_ref[0, 0] / nrm)).astype(jnp.bfloat16)


def _mix(h_ext, part0, W_msg, b_msg, inv_sqrt_lambda):
    nb = 8
    rb = (2 * _N) // nb
    return pl.pallas_call(
        _mix_body,
        grid=(nb,),
        in_specs=[
            pl.BlockSpec((rb, _HW), lambda i: (i, 0)),
            pl.BlockSpec((rb, _HW), lambda i: (i, 0)),
            pl.BlockSpec((_D, _D), lambda i: (0, 0)),
            pl.BlockSpec((_D, _D), lambda i: (0, 0)),
            pl.BlockSpec((1, _D), lambda i: (0, 0)),
            pl.BlockSpec(memory_space=pltpu.SMEM),
        ],
        out_specs=pl.BlockSpec((rb, _D), lambda i: (i, 0)),
        out_shape=jax.ShapeDtypeStruct((2 * _N, _D), jnp.bfloat16),
    )(h_ext, part0, W_msg[:_D], W_msg[_D:], b_msg.reshape(1, _D),
      inv_sqrt_lambda)


_RB = 256
_NBLK = _N // _RB
_G = 16
_Q = 1024


def _sink_body(feats_ref, out_ref, K_ref, u_ref, v_ref):
    i = pl.program_id(0)

    @pl.when(i == 0)
    def _build_and_iterate():
        dT = feats_ref[pl.ds(_N, _N), :].T

        def build(ib, _):
            t = feats_ref[pl.ds(ib * _RB, _RB), :]
            corr = jnp.dot(t, dT, preferred_element_type=jnp.float32)
            Kb = jnp.exp(corr)
            K_ref[pl.ds(ib * _RB, _RB), :] = Kb.astype(jnp.bfloat16)
            rs = jnp.sum(Kb, axis=1, keepdims=True)
            u_ref[pl.ds(ib * _RB, _RB), :] = 0.9 / (rs + 1e-8)
            return 0

        jax.lax.fori_loop(0, _NBLK, build, 0)

        def update_v(_):
            def body(ib, acc):
                Kb = K_ref[pl.ds(ib * _RB, _RB), :].astype(jnp.float32)
                ub = u_ref[pl.ds(ib * _RB, _RB), :]
                return acc + jnp.sum(Kb * ub, axis=0, keepdims=True)

            z = jax.lax.fori_loop(0, _NBLK, body,
                                  jnp.zeros((1, _N), jnp.float32))
            v_ref[...] = 0.9 / (z + 1e-8)

        def update_u(_):
            vrow = v_ref[...]

            def body(ib, _c):
                Kb = K_ref[pl.ds(ib * _RB, _RB), :].astype(jnp.float32)
                y = jnp.sum(Kb * vrow, axis=1, keepdims=True)
                u_ref[pl.ds(ib * _RB, _RB), :] = 0.9 / (y + 1e-8)
                return 0

            jax.lax.fori_loop(0, _NBLK, body, 0)

        def sink_iter(t, _):
            update_v(None)

            @pl.when(t < _SINK - 1)
            def _():
                update_u(None)

            return 0

        jax.lax.fori_loop(0, _SINK, sink_iter, 0)

    Kb = K_ref[pl.ds(i * _RB, _RB), :].astype(jnp.float32)
    out_ref[...] = u_ref[pl.ds(i * _RB, _RB), :] * Kb * v_ref[...]


def _sinkhorn(feats):
    return pl.pallas_call(
        _sink_body,
        grid=(_NBLK,),
        in_specs=[pl.BlockSpec((2 * _N, _D), lambda i: (0, 0))],
        out_specs=pl.BlockSpec((_RB, _N), lambda i: (i, 0)),
        out_shape=jax.ShapeDtypeStruct((_N, _N), jnp.float32),
        scratch_shapes=[
            pltpu.VMEM((_N, _N), jnp.bfloat16),
            pltpu.VMEM((_N, 1), jnp.float32),
            pltpu.VMEM((1, _N), jnp.float32),
        ],
        compiler_params=pltpu.CompilerParams(
            dimension_semantics=("arbitrary",),
            vmem_limit_bytes=64 * 1024 * 1024,
        ),
    )(feats)


def kernel(tra_x, tra_edge_index, tra_batch, det_x, det_edge_index,
           det_batch, W_enc, b_enc, W_msg, b_msg, eplison):
    x_all = jnp.concatenate([tra_x, det_x], axis=0)
    src_r = jnp.concatenate([tra_edge_index[0], det_edge_index[0] + _N]
                            ).reshape(2, 16, _NCH, _CH)
    dst_r = jnp.concatenate([tra_edge_index[1], det_edge_index[1]]
                            ).reshape(2, 16, _NCH, _CH)
    zrow = jnp.zeros((_RPS, _HW), jnp.float32)

    h_ext = _encode(x_all, W_enc, b_enc)

    acc = _segment_sum_sc(h_ext, src_r, dst_r, zrow).reshape(2 * _N, _HW)

    lambd = jnp.exp(eplison[0]) + 0.03
    inv_sqrt_lambda = (1.0 / jnp.sqrt(lambd)).reshape(1, 1)

    feats = _mix(h_ext, acc, W_msg, b_msg, inv_sqrt_lambda)
    return _sinkhorn(feats)


# speedup vs baseline: 3.4925x; 1.1007x over previous
import functools

import jax
import jax.numpy as jnp
from jax import lax
from jax.experimental import pallas as pl
from jax.experimental.pallas import tpu as pltpu
from jax.experimental.pallas import tpu_sc as plsc

_N = 4096
_D = 128
_HW = 144
_SINK = 8

_CH = 128
_NCH = _N * 16 // (16 * _CH)
_RPS = _N // 16


def _enc_body(x_ref, w_ref, b_ref, o_ref):
    h = jnp.dot(x_ref[...], w_ref[...], preferred_element_type=jnp.float32)
    h = jnp.maximum(h + b_ref[...], 0.0)
    o_ref[:, :_D] = h
    lane = jax.lax.broadcasted_iota(jnp.int32, (h.shape[0], _HW - _D), 1)
    o_ref[:, _D:] = jnp.where(lane == 0, 1.0, 0.0)


def _encode(x_all, W_enc, b_enc):
    nb = 8
    rb = (2 * _N) // nb
    return pl.pallas_call(
        _enc_body,
        grid=(nb,),
        in_specs=[
            pl.BlockSpec((rb, _D), lambda i: (i, 0)),
            pl.BlockSpec((_D, _D), lambda i: (0, 0)),
            pl.BlockSpec((1, _D), lambda i: (0, 0)),
        ],
        out_specs=pl.BlockSpec((rb, _HW), lambda i: (i, 0)),
        out_shape=jax.ShapeDtypeStruct((2 * _N, _HW), jnp.float32),
    )(x_all, W_enc, b_enc.reshape(1, _D))


def _seg_body(hext, src, dst, zrow, out, src_v, dst_v, rows_v, acc, sems):
    c = lax.axis_index("c")
    s = lax.axis_index("s")
    pltpu.sync_copy(zrow, acc.at[pl.ds(s * _RPS, _RPS)])
    pltpu.sync_copy(src.at[c, s], src_v)
    pltpu.sync_copy(dst.at[c, s], dst_v)
    plsc.subcore_barrier()

    pltpu.async_copy(hext.at[src_v.at[0]], rows_v.at[0], sems.at[0])

    def chunk(j, carry):
        nxt = j + 1

        @pl.when(nxt < _NCH)
        def _():
            pltpu.async_copy(hext.at[src_v.at[nxt]], rows_v.at[nxt % 2],
                             sems.at[nxt % 2])

        pltpu.make_async_copy(hext.at[src_v.at[j]], rows_v.at[j % 2],
                              sems.at[j % 2]).wait()
        pltpu.sync_copy(rows_v.at[j % 2], acc.at[dst_v.at[j]], add=True)
        return carry

    lax.fori_loop(0, _NCH, chunk, 0)
    plsc.subcore_barrier()
    pltpu.sync_copy(acc.at[pl.ds(s * _RPS, _RPS)],
                    out.at[c, pl.ds(s * _RPS, _RPS)])


def _segment_sum_sc(h_ext, src_r, dst_r, zrow):
    return pl.kernel(
        _seg_body,
        out_type=jax.ShapeDtypeStruct((2, _N, _HW), jnp.float32),
        mesh=plsc.VectorSubcoreMesh(core_axis_name="c", subcore_axis_name="s"),
        scratch_types=[
            pltpu.VMEM((_NCH, _CH), jnp.int32),
            pltpu.VMEM((_NCH, _CH), jnp.int32),
            pltpu.VMEM((2, _CH, _HW), jnp.float32),
            pltpu.VMEM_SHARED((_N, _HW), jnp.float32),
            pltpu.SemaphoreType.DMA((2,)),
        ],
        compiler_params=pltpu.CompilerParams(use_tc_tiling_on_sc=False),
    )(h_ext, src_r, dst_r, zrow)


def _mix_body(hext_ref, p0_ref, wa_ref, wb_ref, b_ref, s_ref, o_ref):
    acc = p0_ref[...]
    deg = jnp.clip(acc[:, _D:_D + 1], 1.0, None)
    msg = acc[:, :_D] / deg
    h = hext_ref[:, :_D]
    o = jnp.dot(h, wa_ref[...], preferred_element_type=jnp.float32)
    o = o + jnp.dot(msg, wb_ref[...], preferred_element_type=jnp.float32)
    o = jnp.maximum(o + b_ref[...], 0.0)
    nrm = jnp.sqrt(jnp.sum(o * o, axis=1, keepdims=True)) + 1e-8
    o_ref[...] = (o * (s_ref[0, 0] / nrm)).astype(jnp.bfloat16)


def _mix(h_ext, part0, W_msg, b_msg, inv_sqrt_lambda):
    nb = 8
    rb = (2 * _N) // nb
    return pl.pallas_call(
        _mix_body,
        grid=(nb,),
        in_specs=[
            pl.BlockSpec((rb, _HW), lambda i: (i, 0)),
            pl.BlockSpec((rb, _HW), lambda i: (i, 0)),
            pl.BlockSpec((_D, _D), lambda i: (0, 0)),
            pl.BlockSpec((_D, _D), lambda i: (0, 0)),
            pl.BlockSpec((1, _D), lambda i: (0, 0)),
            pl.BlockSpec(memory_space=pltpu.SMEM),
        ],
        out_specs=pl.BlockSpec((rb, _D), lambda i: (i, 0)),
        out_shape=jax.ShapeDtypeStruct((2 * _N, _D), jnp.bfloat16),
    )(h_ext, part0, W_msg[:_D], W_msg[_D:], b_msg.reshape(1, _D),
      inv_sqrt_lambda)


_RB = 256
_NBLK = _N // _RB
_G = 16
_Q = 1024


def _sink_body(feats_ref, out_ref, K_ref, u_ref, v_ref):
    i = pl.program_id(0)

    @pl.when(i == 0)
    def _build_and_iterate():
        dT = feats_ref[pl.ds(_N, _N), :].T

        def build(ib, _):
            t = feats_ref[pl.ds(ib * _RB, _RB), :]
            corr = jnp.dot(t, dT, preferred_element_type=jnp.float32)
            Kb = jnp.exp(corr)
            K_ref[pl.ds(ib * _RB, _RB), :] = Kb.astype(jnp.bfloat16)
            rs = jnp.sum(Kb, axis=1, keepdims=True)
            u_ref[pl.ds(ib * _RB, _RB), :] = 0.9 / (rs + 1e-8)
            return 0

        jax.lax.fori_loop(0, _NBLK, build, 0)

        def update_v(_):
            def body(ib, acc):
                Kb = K_ref[pl.ds(ib * _RB, _RB), :].astype(jnp.float32)
                ub = u_ref[pl.ds(ib * _RB, _RB), :]
                return acc + jnp.sum(Kb * ub, axis=0, keepdims=True)

            z = jax.lax.fori_loop(0, _NBLK, body,
                                  jnp.zeros((1, _N), jnp.float32))
            v_ref[...] = 0.9 / (z + 1e-8)

        def update_u(_):
            vrow = v_ref[...]

            def body(ib, _c):
                Kb = K_ref[pl.ds(ib * _RB, _RB), :].astype(jnp.float32)
                y = jnp.sum(Kb * vrow, axis=1, keepdims=True)
                u_ref[pl.ds(ib * _RB, _RB), :] = 0.9 / (y + 1e-8)
                return 0

            jax.lax.fori_loop(0, _NBLK, body, 0)

        def sink_iter(t, _):
            update_v(None)

            @pl.when(t < _SINK - 1)
            def _():
                update_u(None)

            return 0

        jax.lax.fori_loop(0, _SINK, sink_iter, 0)

    Kb = K_ref[pl.ds(i * _RB, _RB), :].astype(jnp.float32)
    out_ref[...] = u_ref[pl.ds(i * _RB, _RB), :] * Kb * v_ref[...]


def _sinkhorn(feats):
    return pl.pallas_call(
        _sink_body,
        grid=(_NBLK,),
        in_specs=[pl.BlockSpec((2 * _N, _D), lambda i: (0, 0))],
        out_specs=pl.BlockSpec((_RB, _N), lambda i: (i, 0)),
        out_shape=jax.ShapeDtypeStruct((_N, _N), jnp.float32),
        scratch_shapes=[
            pltpu.VMEM((_N, _N), jnp.bfloat16),
            pltpu.VMEM((_N, 1), jnp.float32),
            pltpu.VMEM((1, _N), jnp.float32),
        ],
        compiler_params=pltpu.CompilerParams(
            dimension_semantics=("arbitrary",),
            vmem_limit_bytes=64 * 1024 * 1024,
        ),
    )(feats)


def kernel(tra_x, tra_edge_index, tra_batch, det_x, det_edge_index,
           det_batch, W_enc, b_enc, W_msg, b_msg, eplison):
    x_all = jnp.concatenate([tra_x, det_x], axis=0)
    src_r = jnp.concatenate([tra_edge_index[0], det_edge_index[0] + _N]
                            ).reshape(2, 16, _NCH, _CH)
    dst_r = jnp.concatenate([tra_edge_index[1], det_edge_index[1]]
                            ).reshape(2, 16, _NCH, _CH)
    zrow = jnp.zeros((_RPS, _HW), jnp.float32)

    h_ext = _encode(x_all, W_enc, b_enc)

    acc = _segment_sum_sc(h_ext, src_r, dst_r, zrow).reshape(2 * _N, _HW)

    lambd = jnp.exp(eplison[0]) + 0.03
    inv_sqrt_lambda = (1.0 / jnp.sqrt(lambd)).reshape(1, 1)

    feats = _mix(h_ext, acc, W_msg, b_msg, inv_sqrt_lambda)
    return _sinkhorn(feats)


# speedup vs baseline: 3.6602x; 1.0480x over previous
import functools

import jax
import jax.numpy as jnp
from jax import lax
from jax.experimental import pallas as pl
from jax.experimental.pallas import tpu as pltpu
from jax.experimental.pallas import tpu_sc as plsc

_N = 4096
_D = 128
_HW = 144
_SINK = 8

_CH = 128
_NCH = _N * 16 // (16 * _CH)
_RPS = _N // 16


def _enc_body(x_ref, w_ref, b_ref, o_ref):
    h = jnp.dot(x_ref[...], w_ref[...], preferred_element_type=jnp.float32)
    h = jnp.maximum(h + b_ref[...], 0.0)
    o_ref[:, :_D] = h
    lane = jax.lax.broadcasted_iota(jnp.int32, (h.shape[0], _HW - _D), 1)
    o_ref[:, _D:] = jnp.where(lane == 0, 1.0, 0.0)


def _encode(x_all, W_enc, b_enc):
    nb = 8
    rb = (2 * _N) // nb
    return pl.pallas_call(
        _enc_body,
        grid=(nb,),
        in_specs=[
            pl.BlockSpec((rb, _D), lambda i: (i, 0)),
            pl.BlockSpec((_D, _D), lambda i: (0, 0)),
            pl.BlockSpec((1, _D), lambda i: (0, 0)),
        ],
        out_specs=pl.BlockSpec((rb, _HW), lambda i: (i, 0)),
        out_shape=jax.ShapeDtypeStruct((2 * _N, _HW), jnp.float32),
    )(x_all, W_enc, b_enc.reshape(1, _D))


def _seg_body(hext, src, dst, zrow, out, src_v, dst_v, rows_v, acc, sems):
    c = lax.axis_index("c")
    s = lax.axis_index("s")
    pltpu.sync_copy(zrow, acc.at[pl.ds(s * _RPS, _RPS)])
    pltpu.sync_copy(src.at[c, s], src_v)
    pltpu.sync_copy(dst.at[c, s], dst_v)
    plsc.subcore_barrier()

    pltpu.async_copy(hext.at[src_v.at[0]], rows_v.at[0], sems.at[0])

    def chunk(j, carry):
        nxt = j + 1

        @pl.when(nxt < _NCH)
        def _():
            pltpu.async_copy(hext.at[src_v.at[nxt]], rows_v.at[nxt % 2],
                             sems.at[nxt % 2])

        pltpu.make_async_copy(hext.at[src_v.at[j]], rows_v.at[j % 2],
                              sems.at[j % 2]).wait()
        pltpu.sync_copy(rows_v.at[j % 2], acc.at[dst_v.at[j]], add=True)
        return carry

    lax.fori_loop(0, _NCH, chunk, 0)
    plsc.subcore_barrier()
    pltpu.sync_copy(acc.at[pl.ds(s * _RPS, _RPS)],
                    out.at[c, pl.ds(s * _RPS, _RPS)])


def _segment_sum_sc(h_ext, src_r, dst_r, zrow):
    return pl.kernel(
        _seg_body,
        out_type=jax.ShapeDtypeStruct((2, _N, _HW), jnp.float32),
        mesh=plsc.VectorSubcoreMesh(core_axis_name="c", subcore_axis_name="s"),
        scratch_types=[
            pltpu.VMEM((_NCH, _CH), jnp.int32),
            pltpu.VMEM((_NCH, _CH), jnp.int32),
            pltpu.VMEM((2, _CH, _HW), jnp.float32),
            pltpu.VMEM_SHARED((_N, _HW), jnp.float32),
            pltpu.SemaphoreType.DMA((2,)),
        ],
        compiler_params=pltpu.CompilerParams(use_tc_tiling_on_sc=False),
    )(h_ext, src_r, dst_r, zrow)


def _mix_body(hext_ref, p0_ref, wa_ref, wb_ref, b_ref, s_ref, o_ref):
    acc = p0_ref[...]
    deg = jnp.clip(acc[:, _D:_D + 1], 1.0, None)
    msg = acc[:, :_D] / deg
    h = hext_ref[:, :_D]
    o = jnp.dot(h, wa_ref[...], preferred_element_type=jnp.float32)
    o = o + jnp.dot(msg, wb_ref[...], preferred_element_type=jnp.float32)
    o = jnp.maximum(o + b_ref[...], 0.0)
    nrm = jnp.sqrt(jnp.sum(o * o, axis=1, keepdims=True)) + 1e-8
    o_ref[...] = (o * (s_ref[0, 0] / nrm)).astype(jnp.bfloat16)


def _mix(h_ext, part0, W_msg, b_msg, inv_sqrt_lambda):
    nb = 8
    rb = (2 * _N) // nb
    return pl.pallas_call(
        _mix_body,
        grid=(nb,),
        in_specs=[
            pl.BlockSpec((rb, _HW), lambda i: (i, 0)),
            pl.BlockSpec((rb, _HW), lambda i: (i, 0)),
            pl.BlockSpec((_D, _D), lambda i: (0, 0)),
            pl.BlockSpec((_D, _D), lambda i: (0, 0)),
            pl.BlockSpec((1, _D), lambda i: (0, 0)),
            pl.BlockSpec(memory_space=pltpu.SMEM),
        ],
        out_specs=pl.BlockSpec((rb, _D), lambda i: (i, 0)),
        out_shape=jax.ShapeDtypeStruct((2 * _N, _D), jnp.bfloat16),
    )(h_ext, part0, W_msg[:_D], W_msg[_D:], b_msg.reshape(1, _D),
      inv_sqrt_lambda)


_RB = 256
_NBLK = _N // _RB
_G = 16
_Q = 1024


def _sink_body(feats_ref, out_ref, K_ref, u_ref, v_ref):
    i = pl.program_id(0)

    @pl.when(i == 0)
    def _build_and_iterate():
        dT = feats_ref[pl.ds(_N, _N), :].T

        def build(ib, _):
            t = feats_ref[pl.ds(ib * _RB, _RB), :]
            corr = jnp.dot(t, dT, preferred_element_type=jnp.float32)
            Kb = jnp.exp(corr)
            K_ref[pl.ds(ib * _RB, _RB), :] = Kb.astype(jnp.bfloat16)
            rs = jnp.sum(Kb, axis=1, keepdims=True)
            u_ref[pl.ds(ib * _RB, _RB), :] = 0.9 / (rs + 1e-8)
            return 0

        jax.lax.fori_loop(0, _NBLK, build, 0)

        def update_v(_):
            def body(ib, acc):
                Kb = K_ref[pl.ds(ib * _RB, _RB), :]
                ub = u_ref[pl.ds(ib * _RB, _RB), :].astype(jnp.bfloat16)
                return acc + jnp.sum(Kb * ub, axis=0, keepdims=True,
                                     dtype=jnp.float32)

            z = jax.lax.fori_loop(0, _NBLK, body,
                                  jnp.zeros((1, _N), jnp.float32))
            v_ref[...] = 0.9 / (z + 1e-8)

        def update_u(_):
            vrow = v_ref[...].astype(jnp.bfloat16)

            def body(ib, _c):
                Kb = K_ref[pl.ds(ib * _RB, _RB), :]
                y = jnp.sum(Kb * vrow, axis=1, keepdims=True,
                            dtype=jnp.float32)
                u_ref[pl.ds(ib * _RB, _RB), :] = 0.9 / (y + 1e-8)
                return 0

            jax.lax.fori_loop(0, _NBLK, body, 0)

        def sink_iter(t, _):
            update_v(None)

            @pl.when(t < _SINK - 1)
            def _():
                update_u(None)

            return 0

        jax.lax.fori_loop(0, _SINK, sink_iter, 0)

    Kb = K_ref[pl.ds(i * _RB, _RB), :].astype(jnp.float32)
    out_ref[...] = u_ref[pl.ds(i * _RB, _RB), :] * Kb * v_ref[...]


def _sinkhorn(feats):
    return pl.pallas_call(
        _sink_body,
        grid=(_NBLK,),
        in_specs=[pl.BlockSpec((2 * _N, _D), lambda i: (0, 0))],
        out_specs=pl.BlockSpec((_RB, _N), lambda i: (i, 0)),
        out_shape=jax.ShapeDtypeStruct((_N, _N), jnp.float32),
        scratch_shapes=[
            pltpu.VMEM((_N, _N), jnp.bfloat16),
            pltpu.VMEM((_N, 1), jnp.float32),
            pltpu.VMEM((1, _N), jnp.float32),
        ],
        compiler_params=pltpu.CompilerParams(
            dimension_semantics=("arbitrary",),
            vmem_limit_bytes=64 * 1024 * 1024,
        ),
    )(feats)


def kernel(tra_x, tra_edge_index, tra_batch, det_x, det_edge_index,
           det_batch, W_enc, b_enc, W_msg, b_msg, eplison):
    x_all = jnp.concatenate([tra_x, det_x], axis=0)
    src_r = jnp.concatenate([tra_edge_index[0], det_edge_index[0] + _N]
                            ).reshape(2, 16, _NCH, _CH)
    dst_r = jnp.concatenate([tra_edge_index[1], det_edge_index[1]]
                            ).reshape(2, 16, _NCH, _CH)
    zrow = jnp.zeros((_RPS, _HW), jnp.float32)

    h_ext = _encode(x_all, W_enc, b_enc)

    acc = _segment_sum_sc(h_ext, src_r, dst_r, zrow).reshape(2 * _N, _HW)

    lambd = jnp.exp(eplison[0]) + 0.03
    inv_sqrt_lambda = (1.0 / jnp.sqrt(lambd)).reshape(1, 1)

    feats = _mix(h_ext, acc, W_msg, b_msg, inv_sqrt_lambda)
    return _sinkhorn(feats)


# speedup vs baseline: 3.9533x; 1.0801x over previous
import functools

import jax
import jax.numpy as jnp
from jax import lax
from jax.experimental import pallas as pl
from jax.experimental.pallas import tpu as pltpu
from jax.experimental.pallas import tpu_sc as plsc

_N = 4096
_D = 128
_HW = 144
_SINK = 8

_CH = 128
_NCH = _N * 16 // (16 * _CH)
_RPS = _N // 16


def _enc_body(x_ref, w_ref, b_ref, o_ref):
    h = jnp.dot(x_ref[...], w_ref[...], preferred_element_type=jnp.float32)
    h = jnp.maximum(h + b_ref[...], 0.0)
    o_ref[:, :_D] = h
    lane = jax.lax.broadcasted_iota(jnp.int32, (h.shape[0], _HW - _D), 1)
    o_ref[:, _D:] = jnp.where(lane == 0, 1.0, 0.0)


def _encode(x_all, W_enc, b_enc):
    nb = 8
    rb = (2 * _N) // nb
    return pl.pallas_call(
        _enc_body,
        grid=(nb,),
        in_specs=[
            pl.BlockSpec((rb, _D), lambda i: (i, 0)),
            pl.BlockSpec((_D, _D), lambda i: (0, 0)),
            pl.BlockSpec((1, _D), lambda i: (0, 0)),
        ],
        out_specs=pl.BlockSpec((rb, _HW), lambda i: (i, 0)),
        out_shape=jax.ShapeDtypeStruct((2 * _N, _HW), jnp.float32),
    )(x_all, W_enc, b_enc.reshape(1, _D))


def _seg_body(hext, src, dst, zrow, out, src_v, dst_v, rows_v, acc, sems):
    c = lax.axis_index("c")
    s = lax.axis_index("s")
    pltpu.sync_copy(zrow, acc.at[pl.ds(s * _RPS, _RPS)])
    pltpu.sync_copy(src.at[c, s], src_v)
    pltpu.sync_copy(dst.at[c, s], dst_v)
    plsc.subcore_barrier()

    pltpu.async_copy(hext.at[src_v.at[0]], rows_v.at[0], sems.at[0])

    def chunk(j, carry):
        nxt = j + 1

        @pl.when(nxt < _NCH)
        def _():
            pltpu.async_copy(hext.at[src_v.at[nxt]], rows_v.at[nxt % 2],
                             sems.at[nxt % 2])

        pltpu.make_async_copy(hext.at[src_v.at[j]], rows_v.at[j % 2],
                              sems.at[j % 2]).wait()
        pltpu.sync_copy(rows_v.at[j % 2], acc.at[dst_v.at[j]], add=True)
        return carry

    lax.fori_loop(0, _NCH, chunk, 0)
    plsc.subcore_barrier()
    pltpu.sync_copy(acc.at[pl.ds(s * _RPS, _RPS)],
                    out.at[c, pl.ds(s * _RPS, _RPS)])


def _segment_sum_sc(h_ext, src_r, dst_r, zrow):
    return pl.kernel(
        _seg_body,
        out_type=jax.ShapeDtypeStruct((2, _N, _HW), jnp.float32),
        mesh=plsc.VectorSubcoreMesh(core_axis_name="c", subcore_axis_name="s"),
        scratch_types=[
            pltpu.VMEM((_NCH, _CH), jnp.int32),
            pltpu.VMEM((_NCH, _CH), jnp.int32),
            pltpu.VMEM((2, _CH, _HW), jnp.float32),
            pltpu.VMEM_SHARED((_N, _HW), jnp.float32),
            pltpu.SemaphoreType.DMA((2,)),
        ],
        compiler_params=pltpu.CompilerParams(use_tc_tiling_on_sc=False),
    )(h_ext, src_r, dst_r, zrow)


def _mix_body(hext_ref, p0_ref, wa_ref, wb_ref, b_ref, s_ref, o_ref):
    acc = p0_ref[...]
    deg = jnp.clip(acc[:, _D:_D + 1], 1.0, None)
    msg = acc[:, :_D] / deg
    h = hext_ref[:, :_D]
    o = jnp.dot(h, wa_ref[...], preferred_element_type=jnp.float32)
    o = o + jnp.dot(msg, wb_ref[...], preferred_element_type=jnp.float32)
    o = jnp.maximum(o + b_ref[...], 0.0)
    nrm = jnp.sqrt(jnp.sum(o * o, axis=1, keepdims=True)) + 1e-8
    o_ref[...] = (o * (s_ref[0, 0] / nrm)).astype(jnp.bfloat16)


def _mix(h_ext, part0, W_msg, b_msg, inv_sqrt_lambda):
    nb = 8
    rb = (2 * _N) // nb
    return pl.pallas_call(
        _mix_body,
        grid=(nb,),
        in_specs=[
            pl.BlockSpec((rb, _HW), lambda i: (i, 0)),
            pl.BlockSpec((rb, _HW), lambda i: (i, 0)),
            pl.BlockSpec((_D, _D), lambda i: (0, 0)),
            pl.BlockSpec((_D, _D), lambda i: (0, 0)),
            pl.BlockSpec((1, _D), lambda i: (0, 0)),
            pl.BlockSpec(memory_space=pltpu.SMEM),
        ],
        out_specs=pl.BlockSpec((rb, _D), lambda i: (i, 0)),
        out_shape=jax.ShapeDtypeStruct((2 * _N, _D), jnp.bfloat16),
    )(h_ext, part0, W_msg[:_D], W_msg[_D:], b_msg.reshape(1, _D),
      inv_sqrt_lambda)


_RB = 256
_NBLK = _N // _RB
_G = 16
_Q = 1024


def _sink_body(feats_ref, out_ref, K_ref, u_ref, v_ref):
    i = pl.program_id(0)

    @pl.when(i == 0)
    def _build_and_iterate():
        dT = feats_ref[pl.ds(_N, _N), :].T

        def build(ib, zacc):
            t = feats_ref[pl.ds(ib * _RB, _RB), :]
            corr = jnp.dot(t, dT, preferred_element_type=jnp.float32)
            Kb = jnp.exp(corr)
            K_ref[pl.ds(ib * _RB, _RB), :] = Kb.astype(jnp.bfloat16)
            rs = jnp.sum(Kb, axis=1, keepdims=True)
            ub = 0.9 / (rs + 1e-8)
            u_ref[pl.ds(ib * _RB, _RB), :] = ub
            return zacc + jnp.sum(Kb * ub, axis=0, keepdims=True)

        z1 = jax.lax.fori_loop(0, _NBLK, build,
                               jnp.zeros((1, _N), jnp.float32))
        v_ref[...] = 0.9 / (z1 + 1e-8)

        def update_v(_):
            def body(ib, acc):
                Kb = K_ref[pl.ds(ib * _RB, _RB), :]
                ub = u_ref[pl.ds(ib * _RB, _RB), :].astype(jnp.bfloat16)
                prod = Kb * ub
                ph = prod[:_RB // 2] + prod[_RB // 2:]
                return acc + jnp.sum(ph, axis=0, keepdims=True,
                                     dtype=jnp.float32)

            z = jax.lax.fori_loop(0, _NBLK, body,
                                  jnp.zeros((1, _N), jnp.float32))
            v_ref[...] = 0.9 / (z + 1e-8)

        def update_u(_):
            vrow = v_ref[...].astype(jnp.bfloat16)

            def body(ib, _c):
                Kb = K_ref[pl.ds(ib * _RB, _RB), :]
                prod = Kb * vrow
                ph = prod[:, :_N // 2] + prod[:, _N // 2:]
                y = jnp.sum(ph, axis=1, keepdims=True, dtype=jnp.float32)
                u_ref[pl.ds(ib * _RB, _RB), :] = 0.9 / (y + 1e-8)
                return 0

            jax.lax.fori_loop(0, _NBLK, body, 0)

        def sink_iter(t, _):
            update_u(None)
            update_v(None)
            return 0

        jax.lax.fori_loop(0, _SINK - 1, sink_iter, 0)

    Kb = K_ref[pl.ds(i * _RB, _RB), :].astype(jnp.float32)
    out_ref[...] = u_ref[pl.ds(i * _RB, _RB), :] * Kb * v_ref[...]


def _sinkhorn(feats):
    return pl.pallas_call(
        _sink_body,
        grid=(_NBLK,),
        in_specs=[pl.BlockSpec((2 * _N, _D), lambda i: (0, 0))],
        out_specs=pl.BlockSpec((_RB, _N), lambda i: (i, 0)),
        out_shape=jax.ShapeDtypeStruct((_N, _N), jnp.float32),
        scratch_shapes=[
            pltpu.VMEM((_N, _N), jnp.bfloat16),
            pltpu.VMEM((_N, 1), jnp.float32),
            pltpu.VMEM((1, _N), jnp.float32),
        ],
        compiler_params=pltpu.CompilerParams(
            dimension_semantics=("arbitrary",),
            vmem_limit_bytes=64 * 1024 * 1024,
        ),
    )(feats)


def kernel(tra_x, tra_edge_index, tra_batch, det_x, det_edge_index,
           det_batch, W_enc, b_enc, W_msg, b_msg, eplison):
    x_all = jnp.concatenate([tra_x, det_x], axis=0)
    src_r = jnp.concatenate([tra_edge_index[0], det_edge_index[0] + _N]
                            ).reshape(2, 16, _NCH, _CH)
    dst_r = jnp.concatenate([tra_edge_index[1], det_edge_index[1]]
                            ).reshape(2, 16, _NCH, _CH)
    zrow = jnp.zeros((_RPS, _HW), jnp.float32)

    h_ext = _encode(x_all, W_enc, b_enc)

    acc = _segment_sum_sc(h_ext, src_r, dst_r, zrow).reshape(2 * _N, _HW)

    lambd = jnp.exp(eplison[0]) + 0.03
    inv_sqrt_lambda = (1.0 / jnp.sqrt(lambd)).reshape(1, 1)

    feats = _mix(h_ext, acc, W_msg, b_msg, inv_sqrt_lambda)
    return _sinkhorn(feats)


# speedup vs baseline: 4.1422x; 1.0478x over previous
import functools

import jax
import jax.numpy as jnp
from jax import lax
from jax.experimental import pallas as pl
from jax.experimental.pallas import tpu as pltpu
from jax.experimental.pallas import tpu_sc as plsc

_N = 4096
_D = 128
_HW = 144
_SINK = 8

_CH = 128
_NCH = _N * 16 // (16 * _CH)
_RPS = _N // 16


def _enc_body(x_ref, w_ref, b_ref, o_ref):
    h = jnp.dot(x_ref[...], w_ref[...], preferred_element_type=jnp.float32)
    h = jnp.maximum(h + b_ref[...], 0.0)
    o_ref[:, :_D] = h
    lane = jax.lax.broadcasted_iota(jnp.int32, (h.shape[0], _HW - _D), 1)
    o_ref[:, _D:] = jnp.where(lane == 0, 1.0, 0.0)


def _encode(x_all, W_enc, b_enc):
    nb = 8
    rb = (2 * _N) // nb
    return pl.pallas_call(
        _enc_body,
        grid=(nb,),
        in_specs=[
            pl.BlockSpec((rb, _D), lambda i: (i, 0)),
            pl.BlockSpec((_D, _D), lambda i: (0, 0)),
            pl.BlockSpec((1, _D), lambda i: (0, 0)),
        ],
        out_specs=pl.BlockSpec((rb, _HW), lambda i: (i, 0)),
        out_shape=jax.ShapeDtypeStruct((2 * _N, _HW), jnp.float32),
    )(x_all, W_enc, b_enc.reshape(1, _D))


def _seg_body(hext, src, dst, zrow, out, src_v, dst_v, rows_v, acc, sems):
    c = lax.axis_index("c")
    s = lax.axis_index("s")
    pltpu.sync_copy(zrow, acc.at[pl.ds(s * _RPS, _RPS)])
    pltpu.sync_copy(src.at[c, s], src_v)
    pltpu.sync_copy(dst.at[c, s], dst_v)
    plsc.subcore_barrier()

    pltpu.async_copy(hext.at[src_v.at[0]], rows_v.at[0], sems.at[0])

    def chunk(j, carry):
        nxt = j + 1

        @pl.when(nxt < _NCH)
        def _():
            pltpu.async_copy(hext.at[src_v.at[nxt]], rows_v.at[nxt % 2],
                             sems.at[nxt % 2])

        pltpu.make_async_copy(hext.at[src_v.at[j]], rows_v.at[j % 2],
                              sems.at[j % 2]).wait()
        pltpu.sync_copy(rows_v.at[j % 2], acc.at[dst_v.at[j]], add=True)
        return carry

    lax.fori_loop(0, _NCH, chunk, 0)
    plsc.subcore_barrier()
    pltpu.sync_copy(acc.at[pl.ds(s * _RPS, _RPS)],
                    out.at[c, pl.ds(s * _RPS, _RPS)])


def _segment_sum_sc(h_ext, src_r, dst_r, zrow):
    return pl.kernel(
        _seg_body,
        out_type=jax.ShapeDtypeStruct((2, _N, _HW), jnp.float32),
        mesh=plsc.VectorSubcoreMesh(core_axis_name="c", subcore_axis_name="s"),
        scratch_types=[
            pltpu.VMEM((_NCH, _CH), jnp.int32),
            pltpu.VMEM((_NCH, _CH), jnp.int32),
            pltpu.VMEM((2, _CH, _HW), jnp.float32),
            pltpu.VMEM_SHARED((_N, _HW), jnp.float32),
            pltpu.SemaphoreType.DMA((2,)),
        ],
        compiler_params=pltpu.CompilerParams(use_tc_tiling_on_sc=False),
    )(h_ext, src_r, dst_r, zrow)


def _mix_body(hext_ref, p0_ref, wa_ref, wb_ref, b_ref, s_ref, o_ref):
    acc = p0_ref[...]
    deg = jnp.clip(acc[:, _D:_D + 1], 1.0, None)
    msg = acc[:, :_D] / deg
    h = hext_ref[:, :_D]
    o = jnp.dot(h, wa_ref[...], preferred_element_type=jnp.float32)
    o = o + jnp.dot(msg, wb_ref[...], preferred_element_type=jnp.float32)
    o = jnp.maximum(o + b_ref[...], 0.0)
    nrm = jnp.sqrt(jnp.sum(o * o, axis=1, keepdims=True)) + 1e-8
    o_ref[...] = (o * (s_ref[0, 0] / nrm)).astype(jnp.bfloat16)


def _mix(h_ext, part0, W_msg, b_msg, inv_sqrt_lambda):
    nb = 8
    rb = (2 * _N) // nb
    return pl.pallas_call(
        _mix_body,
        grid=(nb,),
        in_specs=[
            pl.BlockSpec((rb, _HW), lambda i: (i, 0)),
            pl.BlockSpec((rb, _HW), lambda i: (i, 0)),
            pl.BlockSpec((_D, _D), lambda i: (0, 0)),
            pl.BlockSpec((_D, _D), lambda i: (0, 0)),
            pl.BlockSpec((1, _D), lambda i: (0, 0)),
            pl.BlockSpec(memory_space=pltpu.SMEM),
        ],
        out_specs=pl.BlockSpec((rb, _D), lambda i: (i, 0)),
        out_shape=jax.ShapeDtypeStruct((2 * _N, _D), jnp.bfloat16),
    )(h_ext, part0, W_msg[:_D], W_msg[_D:], b_msg.reshape(1, _D),
      inv_sqrt_lambda)


_RB = 256
_NBLK = _N // _RB
_G = 16
_Q = 1024


def _sink_body(feats_ref, out_ref, K_ref, u_ref, v_ref):
    i = pl.program_id(0)

    @pl.when(i == 0)
    def _build_and_iterate():
        dT = feats_ref[pl.ds(_N, _N), :].T

        def build(ib, zacc):
            t = feats_ref[pl.ds(ib * _RB, _RB), :]
            corr = jnp.dot(t, dT, preferred_element_type=jnp.float32)
            Kb = jnp.exp(corr)
            K_ref[pl.ds(ib * _RB, _RB), :] = Kb.astype(jnp.bfloat16)
            rs = jnp.sum(Kb, axis=1, keepdims=True)
            ub = 0.9 / (rs + 1e-8)
            u_ref[pl.ds(ib * _RB, _RB), :] = ub
            return zacc + jnp.sum(Kb * ub, axis=0, keepdims=True)

        z1 = jax.lax.fori_loop(0, _NBLK, build,
                               jnp.zeros((1, _N), jnp.float32))
        v_ref[...] = 0.9 / (z1 + 1e-8)

        def update_v(_):
            def body(ib, acc):
                Kb = K_ref[pl.ds(ib * _RB, _RB), :]
                ub = u_ref[pl.ds(ib * _RB, _RB), :].astype(jnp.bfloat16)
                prod = Kb * ub
                ph = prod[:_RB // 2] + prod[_RB // 2:]
                ph = ph[:_RB // 4] + ph[_RB // 4:]
                return acc + jnp.sum(ph, axis=0, keepdims=True,
                                     dtype=jnp.float32)

            z = jax.lax.fori_loop(0, _NBLK, body,
                                  jnp.zeros((1, _N), jnp.float32))
            v_ref[...] = 0.9 / (z + 1e-8)

        def update_u(_):
            vrow = v_ref[...].astype(jnp.bfloat16)

            def body(ib, _c):
                Kb = K_ref[pl.ds(ib * _RB, _RB), :]
                prod = Kb * vrow
                ph = prod[:, :_N // 2] + prod[:, _N // 2:]
                ph = ph[:, :_N // 4] + ph[:, _N // 4:]
                y = jnp.sum(ph, axis=1, keepdims=True, dtype=jnp.float32)
                u_ref[pl.ds(ib * _RB, _RB), :] = 0.9 / (y + 1e-8)
                return 0

            jax.lax.fori_loop(0, _NBLK, body, 0)

        def sink_iter(t, _):
            update_u(None)
            update_v(None)
            return 0

        jax.lax.fori_loop(0, _SINK - 1, sink_iter, 0)

    Kb = K_ref[pl.ds(i * _RB, _RB), :]
    ub = u_ref[pl.ds(i * _RB, _RB), :].astype(jnp.bfloat16)
    vb = v_ref[...].astype(jnp.bfloat16)
    out_ref[...] = ((ub * Kb) * vb).astype(jnp.float32)


def _sinkhorn(feats):
    return pl.pallas_call(
        _sink_body,
        grid=(_NBLK,),
        in_specs=[pl.BlockSpec((2 * _N, _D), lambda i: (0, 0))],
        out_specs=pl.BlockSpec((_RB, _N), lambda i: (i, 0)),
        out_shape=jax.ShapeDtypeStruct((_N, _N), jnp.float32),
        scratch_shapes=[
            pltpu.VMEM((_N, _N), jnp.bfloat16),
            pltpu.VMEM((_N, 1), jnp.float32),
            pltpu.VMEM((1, _N), jnp.float32),
        ],
        compiler_params=pltpu.CompilerParams(
            dimension_semantics=("arbitrary",),
            vmem_limit_bytes=64 * 1024 * 1024,
        ),
    )(feats)


def kernel(tra_x, tra_edge_index, tra_batch, det_x, det_edge_index,
           det_batch, W_enc, b_enc, W_msg, b_msg, eplison):
    x_all = jnp.concatenate([tra_x, det_x], axis=0)
    src_r = jnp.concatenate([tra_edge_index[0], det_edge_index[0] + _N]
                            ).reshape(2, 16, _NCH, _CH)
    dst_r = jnp.concatenate([tra_edge_index[1], det_edge_index[1]]
                            ).reshape(2, 16, _NCH, _CH)
    zrow = jnp.zeros((_RPS, _HW), jnp.float32)

    h_ext = _encode(x_all, W_enc, b_enc)

    acc = _segment_sum_sc(h_ext, src_r, dst_r, zrow).reshape(2 * _N, _HW)

    lambd = jnp.exp(eplison[0]) + 0.03
    inv_sqrt_lambda = (1.0 / jnp.sqrt(lambd)).reshape(1, 1)

    feats = _mix(h_ext, acc, W_msg, b_msg, inv_sqrt_lambda)
    return _sinkhorn(feats)


# speedup vs baseline: 4.2290x; 1.0210x over previous
import functools

import jax
import jax.numpy as jnp
from jax import lax
from jax.experimental import pallas as pl
from jax.experimental.pallas import tpu as pltpu
from jax.experimental.pallas import tpu_sc as plsc

_N = 4096
_D = 128
_HW = 144
_SINK = 8

_CH = 128
_NCH = _N * 16 // (16 * _CH)
_RPS = _N // 16


def _enc_body(x_ref, w_ref, b_ref, o_ref):
    h = jnp.dot(x_ref[...], w_ref[...], preferred_element_type=jnp.float32)
    h = jnp.maximum(h + b_ref[...], 0.0)
    o_ref[:, :_D] = h
    lane = jax.lax.broadcasted_iota(jnp.int32, (h.shape[0], _HW - _D), 1)
    o_ref[:, _D:] = jnp.where(lane == 0, 1.0, 0.0)


def _encode(x_all, W_enc, b_enc):
    nb = 8
    rb = (2 * _N) // nb
    return pl.pallas_call(
        _enc_body,
        grid=(nb,),
        in_specs=[
            pl.BlockSpec((rb, _D), lambda i: (i, 0)),
            pl.BlockSpec((_D, _D), lambda i: (0, 0)),
            pl.BlockSpec((1, _D), lambda i: (0, 0)),
        ],
        out_specs=pl.BlockSpec((rb, _HW), lambda i: (i, 0)),
        out_shape=jax.ShapeDtypeStruct((2 * _N, _HW), jnp.float32),
    )(x_all, W_enc, b_enc.reshape(1, _D))


def _seg_body(hext, src, dst, zrow, out, src_v, dst_v, rows_v, acc, sems):
    c = lax.axis_index("c")
    s = lax.axis_index("s")
    pltpu.sync_copy(zrow, acc.at[pl.ds(s * _RPS, _RPS)])
    pltpu.sync_copy(src.at[c, s], src_v)
    pltpu.sync_copy(dst.at[c, s], dst_v)
    plsc.subcore_barrier()

    pltpu.async_copy(hext.at[src_v.at[0]], rows_v.at[0], sems.at[0])

    def chunk(j, carry):
        nxt = j + 1

        @pl.when(nxt < _NCH)
        def _():
            pltpu.async_copy(hext.at[src_v.at[nxt]], rows_v.at[nxt % 2],
                             sems.at[nxt % 2])

        pltpu.make_async_copy(hext.at[src_v.at[j]], rows_v.at[j % 2],
                              sems.at[j % 2]).wait()
        pltpu.sync_copy(rows_v.at[j % 2], acc.at[dst_v.at[j]], add=True)
        return carry

    lax.fori_loop(0, _NCH, chunk, 0)
    plsc.subcore_barrier()
    pltpu.sync_copy(acc.at[pl.ds(s * _RPS, _RPS)],
                    out.at[c, pl.ds(s * _RPS, _RPS)])


def _segment_sum_sc(h_ext, src_r, dst_r, zrow):
    return pl.kernel(
        _seg_body,
        out_type=jax.ShapeDtypeStruct((2, _N, _HW), jnp.float32),
        mesh=plsc.VectorSubcoreMesh(core_axis_name="c", subcore_axis_name="s"),
        scratch_types=[
            pltpu.VMEM((_NCH, _CH), jnp.int32),
            pltpu.VMEM((_NCH, _CH), jnp.int32),
            pltpu.VMEM((2, _CH, _HW), jnp.float32),
            pltpu.VMEM_SHARED((_N, _HW), jnp.float32),
            pltpu.SemaphoreType.DMA((2,)),
        ],
        compiler_params=pltpu.CompilerParams(use_tc_tiling_on_sc=False),
    )(h_ext, src_r, dst_r, zrow)


def _mix_body(hext_ref, p0_ref, wa_ref, wb_ref, b_ref, s_ref, o_ref):
    acc = p0_ref[...]
    deg = jnp.clip(acc[:, _D:_D + 1], 1.0, None)
    msg = acc[:, :_D] / deg
    h = hext_ref[:, :_D]
    o = jnp.dot(h, wa_ref[...], preferred_element_type=jnp.float32)
    o = o + jnp.dot(msg, wb_ref[...], preferred_element_type=jnp.float32)
    o = jnp.maximum(o + b_ref[...], 0.0)
    nrm = jnp.sqrt(jnp.sum(o * o, axis=1, keepdims=True)) + 1e-8
    o_ref[...] = (o * (s_ref[0, 0] / nrm)).astype(jnp.bfloat16)


def _mix(h_ext, part0, W_msg, b_msg, inv_sqrt_lambda):
    nb = 8
    rb = (2 * _N) // nb
    return pl.pallas_call(
        _mix_body,
        grid=(nb,),
        in_specs=[
            pl.BlockSpec((rb, _HW), lambda i: (i, 0)),
            pl.BlockSpec((rb, _HW), lambda i: (i, 0)),
            pl.BlockSpec((_D, _D), lambda i: (0, 0)),
            pl.BlockSpec((_D, _D), lambda i: (0, 0)),
            pl.BlockSpec((1, _D), lambda i: (0, 0)),
            pl.BlockSpec(memory_space=pltpu.SMEM),
        ],
        out_specs=pl.BlockSpec((rb, _D), lambda i: (i, 0)),
        out_shape=jax.ShapeDtypeStruct((2 * _N, _D), jnp.bfloat16),
    )(h_ext, part0, W_msg[:_D], W_msg[_D:], b_msg.reshape(1, _D),
      inv_sqrt_lambda)


_RB = 256
_NBLK = _N // _RB
_G = 16
_Q = 1024


def _sink_body(feats_ref, out_ref, K_ref, u_ref, v_ref):
    i = pl.program_id(0)

    @pl.when(i == 0)
    def _build_and_iterate():
        dT = feats_ref[pl.ds(_N, _N), :].T

        def build(ib, zacc):
            t = feats_ref[pl.ds(ib * _RB, _RB), :]
            corr = jnp.dot(t, dT, preferred_element_type=jnp.float32)
            Kb = jnp.exp(corr)
            K_ref[pl.ds(ib * _RB, _RB), :] = Kb.astype(jnp.bfloat16)
            rs = jnp.sum(Kb, axis=1, keepdims=True)
            ub = 0.9 / (rs + 1e-8)
            u_ref[pl.ds(ib * _RB, _RB), :] = ub
            return zacc + jnp.sum(Kb * ub, axis=0, keepdims=True)

        z1 = jax.lax.fori_loop(0, _NBLK, build,
                               jnp.zeros((1, _N), jnp.float32))
        v_ref[...] = 0.9 / (z1 + 1e-8)

        def update_v(_):
            def body(ib, acc):
                r = ib * _RB
                ph = None
                for k in range(4):
                    Kp = K_ref[pl.ds(r + k * (_RB // 4), _RB // 4), :]
                    up = u_ref[pl.ds(r + k * (_RB // 4), _RB // 4),
                               :].astype(jnp.bfloat16)
                    t = Kp * up
                    ph = t if ph is None else ph + t
                return acc + jnp.sum(ph, axis=0, keepdims=True,
                                     dtype=jnp.float32)

            z = jax.lax.fori_loop(0, _NBLK, body,
                                  jnp.zeros((1, _N), jnp.float32))
            v_ref[...] = 0.9 / (z + 1e-8)

        def update_u(_):
            vrow = v_ref[...].astype(jnp.bfloat16)

            def body(ib, _c):
                r = ib * _RB
                tot = None
                for k in range(4):
                    Kp = K_ref[pl.ds(r, _RB),
                               k * (_N // 4):(k + 1) * (_N // 4)]
                    vp = vrow[:, k * (_N // 4):(k + 1) * (_N // 4)]
                    t = Kp * vp
                    tot = t if tot is None else tot + t
                y = jnp.sum(tot, axis=1, keepdims=True, dtype=jnp.float32)
                u_ref[pl.ds(ib * _RB, _RB), :] = 0.9 / (y + 1e-8)
                return 0

            jax.lax.fori_loop(0, _NBLK, body, 0)

        def sink_iter(t, _):
            update_u(None)
            update_v(None)
            return 0

        jax.lax.fori_loop(0, _SINK - 1, sink_iter, 0)

    Kb = K_ref[pl.ds(i * _RB, _RB), :]
    ub = u_ref[pl.ds(i * _RB, _RB), :].astype(jnp.bfloat16)
    vb = v_ref[...].astype(jnp.bfloat16)
    out_ref[...] = ((ub * Kb) * vb).astype(jnp.float32)


def _sinkhorn(feats):
    return pl.pallas_call(
        _sink_body,
        grid=(_NBLK,),
        in_specs=[pl.BlockSpec((2 * _N, _D), lambda i: (0, 0))],
        out_specs=pl.BlockSpec((_RB, _N), lambda i: (i, 0)),
        out_shape=jax.ShapeDtypeStruct((_N, _N), jnp.float32),
        scratch_shapes=[
            pltpu.VMEM((_N, _N), jnp.bfloat16),
            pltpu.VMEM((_N, 1), jnp.float32),
            pltpu.VMEM((1, _N), jnp.float32),
        ],
        compiler_params=pltpu.CompilerParams(
            dimension_semantics=("arbitrary",),
            vmem_limit_bytes=64 * 1024 * 1024,
        ),
    )(feats)


def kernel(tra_x, tra_edge_index, tra_batch, det_x, det_edge_index,
           det_batch, W_enc, b_enc, W_msg, b_msg, eplison):
    x_all = jnp.concatenate([tra_x, det_x], axis=0)
    src_r = jnp.concatenate([tra_edge_index[0], det_edge_index[0] + _N]
                            ).reshape(2, 16, _NCH, _CH)
    dst_r = jnp.concatenate([tra_edge_index[1], det_edge_index[1]]
                            ).reshape(2, 16, _NCH, _CH)
    zrow = jnp.zeros((_RPS, _HW), jnp.float32)

    h_ext = _encode(x_all, W_enc, b_enc)

    acc = _segment_sum_sc(h_ext, src_r, dst_r, zrow).reshape(2 * _N, _HW)

    lambd = jnp.exp(eplison[0]) + 0.03
    inv_sqrt_lambda = (1.0 / jnp.sqrt(lambd)).reshape(1, 1)

    feats = _mix(h_ext, acc, W_msg, b_msg, inv_sqrt_lambda)
    return _sinkhorn(feats)


# speedup vs baseline: 4.4587x; 1.0543x over previous
import functools

import jax
import jax.numpy as jnp
from jax import lax
from jax.experimental import pallas as pl
from jax.experimental.pallas import tpu as pltpu
from jax.experimental.pallas import tpu_sc as plsc

_N = 4096
_D = 128
_HW = 144
_SINK = 8

_CH = 128
_NCH = _N * 16 // (16 * _CH)
_RPS = _N // 16


def _enc_body(x_ref, w_ref, b_ref, o_ref):
    h = jnp.dot(x_ref[...], w_ref[...], preferred_element_type=jnp.float32)
    h = jnp.maximum(h + b_ref[...], 0.0)
    o_ref[:, :_D] = h
    lane = jax.lax.broadcasted_iota(jnp.int32, (h.shape[0], _HW - _D), 1)
    o_ref[:, _D:] = jnp.where(lane == 0, 1.0, 0.0)


def _encode(x_all, W_enc, b_enc):
    nb = 8
    rb = (2 * _N) // nb
    return pl.pallas_call(
        _enc_body,
        grid=(nb,),
        in_specs=[
            pl.BlockSpec((rb, _D), lambda i: (i, 0)),
            pl.BlockSpec((_D, _D), lambda i: (0, 0)),
            pl.BlockSpec((1, _D), lambda i: (0, 0)),
        ],
        out_specs=pl.BlockSpec((rb, _HW), lambda i: (i, 0)),
        out_shape=jax.ShapeDtypeStruct((2 * _N, _HW), jnp.float32),
    )(x_all, W_enc, b_enc.reshape(1, _D))


def _seg_body(hext, src, dst, zrow, out, src_v, dst_v, rows_v, acc, sems):
    c = lax.axis_index("c")
    s = lax.axis_index("s")
    pltpu.sync_copy(zrow, acc.at[pl.ds(s * _RPS, _RPS)])
    pltpu.sync_copy(src.at[c, s], src_v)
    pltpu.sync_copy(dst.at[c, s], dst_v)
    plsc.subcore_barrier()

    pltpu.async_copy(hext.at[src_v.at[0]], rows_v.at[0], sems.at[0])

    def chunk(j, carry):
        nxt = j + 1

        @pl.when(nxt < _NCH)
        def _():
            pltpu.async_copy(hext.at[src_v.at[nxt]], rows_v.at[nxt % 2],
                             sems.at[nxt % 2])

        pltpu.make_async_copy(hext.at[src_v.at[j]], rows_v.at[j % 2],
                              sems.at[j % 2]).wait()
        pltpu.sync_copy(rows_v.at[j % 2], acc.at[dst_v.at[j]], add=True)
        return carry

    lax.fori_loop(0, _NCH, chunk, 0)
    plsc.subcore_barrier()
    pltpu.sync_copy(acc.at[pl.ds(s * _RPS, _RPS)],
                    out.at[c, pl.ds(s * _RPS, _RPS)])


def _segment_sum_sc(h_ext, src_r, dst_r, zrow):
    return pl.kernel(
        _seg_body,
        out_type=jax.ShapeDtypeStruct((2, _N, _HW), jnp.float32),
        mesh=plsc.VectorSubcoreMesh(core_axis_name="c", subcore_axis_name="s"),
        scratch_types=[
            pltpu.VMEM((_NCH, _CH), jnp.int32),
            pltpu.VMEM((_NCH, _CH), jnp.int32),
            pltpu.VMEM((2, _CH, _HW), jnp.float32),
            pltpu.VMEM_SHARED((_N, _HW), jnp.float32),
            pltpu.SemaphoreType.DMA((2,)),
        ],
        compiler_params=pltpu.CompilerParams(use_tc_tiling_on_sc=False),
    )(h_ext, src_r, dst_r, zrow)


def _mix_body(hext_ref, p0_ref, wa_ref, wb_ref, b_ref, s_ref, o_ref):
    acc = p0_ref[...]
    deg = jnp.clip(acc[:, _D:_D + 1], 1.0, None)
    msg = acc[:, :_D] / deg
    h = hext_ref[:, :_D]
    o = jnp.dot(h, wa_ref[...], preferred_element_type=jnp.float32)
    o = o + jnp.dot(msg, wb_ref[...], preferred_element_type=jnp.float32)
    o = jnp.maximum(o + b_ref[...], 0.0)
    nrm = jnp.sqrt(jnp.sum(o * o, axis=1, keepdims=True)) + 1e-8
    o_ref[...] = (o * (s_ref[0, 0] / nrm)).astype(jnp.bfloat16)


def _mix(h_ext, part0, W_msg, b_msg, inv_sqrt_lambda):
    nb = 8
    rb = (2 * _N) // nb
    return pl.pallas_call(
        _mix_body,
        grid=(nb,),
        in_specs=[
            pl.BlockSpec((rb, _HW), lambda i: (i, 0)),
            pl.BlockSpec((rb, _HW), lambda i: (i, 0)),
            pl.BlockSpec((_D, _D), lambda i: (0, 0)),
            pl.BlockSpec((_D, _D), lambda i: (0, 0)),
            pl.BlockSpec((1, _D), lambda i: (0, 0)),
            pl.BlockSpec(memory_space=pltpu.SMEM),
        ],
        out_specs=pl.BlockSpec((rb, _D), lambda i: (i, 0)),
        out_shape=jax.ShapeDtypeStruct((2 * _N, _D), jnp.bfloat16),
    )(h_ext, part0, W_msg[:_D], W_msg[_D:], b_msg.reshape(1, _D),
      inv_sqrt_lambda)


_RB = 256
_NBLK = _N // _RB
_OB = 512
_G = 16
_Q = 1024


def _sink_body(feats_ref, out_ref, K_ref, u_ref, v_ref):
    i = pl.program_id(0)

    @pl.when(i == 0)
    def _build_and_iterate():
        dT = feats_ref[pl.ds(_N, _N), :].T

        def build(ib, zacc):
            t = feats_ref[pl.ds(ib * _RB, _RB), :]
            corr = jnp.dot(t, dT, preferred_element_type=jnp.float32)
            Kb = jnp.exp(corr)
            K_ref[pl.ds(ib * _RB, _RB), :] = Kb.astype(jnp.bfloat16)
            rs = jnp.sum(Kb, axis=1, keepdims=True)
            ub = 0.9 / (rs + 1e-8)
            u_ref[pl.ds(ib * _RB, _RB), :] = ub
            return zacc + jnp.sum(Kb * ub, axis=0, keepdims=True)

        z1 = jax.lax.fori_loop(0, _NBLK, build,
                               jnp.zeros((1, _N), jnp.float32))
        v_ref[...] = 0.9 / (z1 + 1e-8)

        def update_v(_):
            def body(ib, acc):
                r = ib * _RB
                ph = None
                for k in range(8):
                    Kp = K_ref[pl.ds(r + k * (_RB // 8), _RB // 8), :]
                    up = u_ref[pl.ds(r + k * (_RB // 8), _RB // 8),
                               :].astype(jnp.bfloat16)
                    t = Kp * up
                    ph = t if ph is None else ph + t
                return acc + jnp.sum(ph, axis=0, keepdims=True,
                                     dtype=jnp.float32)

            z = jax.lax.fori_loop(0, _NBLK, body,
                                  jnp.zeros((1, _N), jnp.float32))
            v_ref[...] = 0.9 / (z + 1e-8)

        def update_u(_):
            vrow = v_ref[...].astype(jnp.bfloat16)

            def body(ib, _c):
                r = ib * _RB
                tot = None
                for k in range(8):
                    Kp = K_ref[pl.ds(r, _RB),
                               k * (_N // 8):(k + 1) * (_N // 8)]
                    vp = vrow[:, k * (_N // 8):(k + 1) * (_N // 8)]
                    t = Kp * vp
                    tot = t if tot is None else tot + t
                y = jnp.sum(tot, axis=1, keepdims=True, dtype=jnp.float32)
                u_ref[pl.ds(ib * _RB, _RB), :] = 0.9 / (y + 1e-8)
                return 0

            jax.lax.fori_loop(0, _NBLK, body, 0)

        def sink_iter(t, _):
            update_u(None)
            update_v(None)
            return 0

        jax.lax.fori_loop(0, _SINK - 1, sink_iter, 0)

    Kb = K_ref[pl.ds(i * _OB, _OB), :]
    ub = u_ref[pl.ds(i * _OB, _OB), :].astype(jnp.bfloat16)
    vb = v_ref[...].astype(jnp.bfloat16)
    out_ref[...] = ((ub * Kb) * vb).astype(jnp.float32)


def _sinkhorn(feats):
    return pl.pallas_call(
        _sink_body,
        grid=(_N // _OB,),
        in_specs=[pl.BlockSpec((2 * _N, _D), lambda i: (0, 0))],
        out_specs=pl.BlockSpec((_OB, _N), lambda i: (i, 0)),
        out_shape=jax.ShapeDtypeStruct((_N, _N), jnp.float32),
        scratch_shapes=[
            pltpu.VMEM((_N, _N), jnp.bfloat16),
            pltpu.VMEM((_N, 1), jnp.float32),
            pltpu.VMEM((1, _N), jnp.float32),
        ],
        compiler_params=pltpu.CompilerParams(
            dimension_semantics=("arbitrary",),
            vmem_limit_bytes=64 * 1024 * 1024,
        ),
    )(feats)


def kernel(tra_x, tra_edge_index, tra_batch, det_x, det_edge_index,
           det_batch, W_enc, b_enc, W_msg, b_msg, eplison):
    x_all = jnp.concatenate([tra_x, det_x], axis=0)
    src_r = jnp.concatenate([tra_edge_index[0], det_edge_index[0] + _N]
                            ).reshape(2, 16, _NCH, _CH)
    dst_r = jnp.concatenate([tra_edge_index[1], det_edge_index[1]]
                            ).reshape(2, 16, _NCH, _CH)
    zrow = jnp.zeros((_RPS, _HW), jnp.float32)

    h_ext = _encode(x_all, W_enc, b_enc)

    acc = _segment_sum_sc(h_ext, src_r, dst_r, zrow).reshape(2 * _N, _HW)

    lambd = jnp.exp(eplison[0]) + 0.03
    inv_sqrt_lambda = (1.0 / jnp.sqrt(lambd)).reshape(1, 1)

    feats = _mix(h_ext, acc, W_msg, b_msg, inv_sqrt_lambda)
    return _sinkhorn(feats)
